# baseline (speedup 1.0000x reference)
import functools

import jax
import jax.numpy as jnp
from jax import lax
from jax.experimental import pallas as pl
from jax.experimental.pallas import tpu as pltpu
from jax.experimental.pallas import tpu_sc as plsc

_VOCAB = 22
_HIDDEN = 2048
_EPS = 1e-12

_B = 4 * 2048
_NC, _NS = 2, 16
_NW = _NC * _NS
_BPW = _B // _NW
_C = 32
_NCHUNK = _BPW // _C


def _norm_body(tab_ref, gamma_ref, beta_ref, out_ref):
    t = tab_ref[...]
    mean = jnp.mean(t, axis=1, keepdims=True)
    c = t - mean
    var = jnp.mean(c * c, axis=1, keepdims=True)
    out_ref[...] = c * lax.rsqrt(var + _EPS) * gamma_ref[...] + beta_ref[...]


def _normalize_table(table, gamma, beta):
    return pl.pallas_call(
        _norm_body,
        out_shape=jax.ShapeDtypeStruct((_VOCAB, _HIDDEN), jnp.float32),
    )(table, gamma.reshape(1, _HIDDEN), beta.reshape(1, _HIDDEN))


_sc_mesh = plsc.VectorSubcoreMesh(core_axis_name="c", subcore_axis_name="s")


@functools.partial(
    pl.kernel,
    mesh=_sc_mesh,
    out_type=jax.ShapeDtypeStruct((_B, _HIDDEN), jnp.float32),
    scratch_types=[
        pltpu.VMEM((_BPW,), jnp.int32),
        pltpu.VMEM((_C, _HIDDEN), jnp.float32),
        pltpu.SemaphoreType.DMA,
    ],
)
def _sc_gather(idx_hbm, tab_hbm, out_hbm, idx_v, rows_v, sem):
    wid = lax.axis_index("s") * _NC + lax.axis_index("c")
    base = wid * _BPW
    pltpu.sync_copy(idx_hbm.at[pl.ds(base, _BPW)], idx_v)
    for i in range(_NCHUNK):
        off = i * _C
        pltpu.async_copy(tab_hbm.at[idx_v.at[pl.ds(off, _C)]], rows_v, sem).wait()
        pltpu.sync_copy(rows_v, out_hbm.at[pl.ds(base + off, _C)])


def kernel(seq, table, gamma, beta):
    norm_tab = _normalize_table(table, gamma, beta)
    idx = seq.reshape(-1).astype(jnp.int32)
    out = _sc_gather(idx, norm_tab)
    return out.reshape(seq.shape[0], seq.shape[1], _HIDDEN)


# speedup vs baseline: 1.0103x; 1.0103x over previous
import functools

import jax
import jax.numpy as jnp
from jax import lax
from jax.experimental import pallas as pl
from jax.experimental.pallas import tpu as pltpu
from jax.experimental.pallas import tpu_sc as plsc

_VOCAB = 22
_HIDDEN = 2048
_EPS = 1e-12

_B = 4 * 2048
_NC, _NS = 2, 16
_NW = _NC * _NS
_BPW = _B // _NW
_C = 16
_NCHUNK = _BPW // _C
_NBUF = 3


def _norm_body(tab_ref, gamma_ref, beta_ref, out_ref):
    t = tab_ref[...]
    mean = jnp.mean(t, axis=1, keepdims=True)
    c = t - mean
    var = jnp.mean(c * c, axis=1, keepdims=True)
    out_ref[...] = c * lax.rsqrt(var + _EPS) * gamma_ref[...] + beta_ref[...]


def _normalize_table(table, gamma, beta):
    return pl.pallas_call(
        _norm_body,
        out_shape=jax.ShapeDtypeStruct((_VOCAB, _HIDDEN), jnp.float32),
    )(table, gamma.reshape(1, _HIDDEN), beta.reshape(1, _HIDDEN))


_sc_mesh = plsc.VectorSubcoreMesh(core_axis_name="c", subcore_axis_name="s")


@functools.partial(
    pl.kernel,
    mesh=_sc_mesh,
    out_type=jax.ShapeDtypeStruct((_B, _HIDDEN), jnp.float32),
    scratch_types=(
        [pltpu.VMEM((_BPW,), jnp.int32)]
        + [pltpu.VMEM((_C, _HIDDEN), jnp.float32) for _ in range(_NBUF)]
        + [pltpu.SemaphoreType.DMA for _ in range(2 * _NBUF)]
    ),
)
def _sc_gather(idx_hbm, tab_hbm, out_hbm, idx_v, *bufs_and_sems):
    rows = bufs_and_sems[:_NBUF]
    gsem = bufs_and_sems[_NBUF:2 * _NBUF]
    wsem = bufs_and_sems[2 * _NBUF:]
    wid = lax.axis_index("s") * _NC + lax.axis_index("c")
    base = wid * _BPW
    pltpu.sync_copy(idx_hbm.at[pl.ds(base, _BPW)], idx_v)
    gather = [None] * _NBUF
    write = [None] * _NBUF
    for i in range(_NCHUNK + 1):
        if i < _NCHUNK:
            b = i % _NBUF
            if write[b] is not None:
                write[b].wait()
            gather[b] = pltpu.async_copy(
                tab_hbm.at[idx_v.at[pl.ds(i * _C, _C)]], rows[b], gsem[b])
        if i >= 1:
            pb = (i - 1) % _NBUF
            gather[pb].wait()
            write[pb] = pltpu.async_copy(
                rows[pb], out_hbm.at[pl.ds(base + (i - 1) * _C, _C)], wsem[pb])
    for b in range(_NBUF):
        if write[b] is not None:
            write[b].wait()


def kernel(seq, table, gamma, beta):
    norm_tab = _normalize_table(table, gamma, beta)
    idx = seq.reshape(-1).astype(jnp.int32)
    out = _sc_gather(idx, norm_tab)
    return out.reshape(seq.shape[0], seq.shape[1], _HIDDEN)


# speedup vs baseline: 2.6336x; 2.6069x over previous
import functools

import jax
import jax.numpy as jnp
from jax import lax
from jax.experimental import pallas as pl
from jax.experimental.pallas import tpu as pltpu
from jax.experimental.pallas import tpu_sc as plsc

_VOCAB = 22
_HIDDEN = 2048
_EPS = 1e-12

_B = 4 * 2048
_NC, _NS = 2, 16
_NW = _NC * _NS
_BPW = _B // _NW
_LANES = 16
_K = 32


def _norm_body(tab_ref, gamma_ref, beta_ref, out_ref):
    t = tab_ref[...]
    mean = jnp.mean(t, axis=1, keepdims=True)
    c = t - mean
    var = jnp.mean(c * c, axis=1, keepdims=True)
    out_ref[...] = c * lax.rsqrt(var + _EPS) * gamma_ref[...] + beta_ref[...]


def _normalize_table(table, gamma, beta):
    return pl.pallas_call(
        _norm_body,
        out_shape=jax.ShapeDtypeStruct((_VOCAB, _HIDDEN), jnp.float32),
    )(table, gamma.reshape(1, _HIDDEN), beta.reshape(1, _HIDDEN))


_sc_mesh = plsc.VectorSubcoreMesh(core_axis_name="c", subcore_axis_name="s")


@functools.partial(
    pl.kernel,
    mesh=_sc_mesh,
    out_type=jax.ShapeDtypeStruct((_B, _HIDDEN), jnp.float32),
    scratch_types=(
        [pltpu.VMEM((_BPW,), jnp.int32),
         pltpu.VMEM((_VOCAB, _HIDDEN), jnp.float32)]
        + [pltpu.SemaphoreType.DMA]
    ),
)
def _sc_scatter_rows(idx_hbm, tab_hbm, out_hbm, idx_v, tab_v, sem):
    wid = lax.axis_index("s") * _NC + lax.axis_index("c")
    base = wid * _BPW
    pltpu.sync_copy(idx_hbm.at[pl.ds(base, _BPW)], idx_v)
    pltpu.sync_copy(tab_hbm, tab_v)

    def group(g, carry):
        descs = []
        for half in range(_K // _LANES):
            iv = idx_v[pl.ds((g * _K + half * _LANES) * 1, _LANES)]
            for b in range(_LANES):
                r = g * _K + half * _LANES + b
                descs.append(pltpu.async_copy(
                    tab_v.at[pl.ds(iv[b], 1)],
                    out_hbm.at[pl.ds(base + r, 1)],
                    sem))
        for d in descs:
            d.wait()
        return carry

    lax.fori_loop(0, _BPW // _K, group, 0)


def kernel(seq, table, gamma, beta):
    norm_tab = _normalize_table(table, gamma, beta)
    idx = seq.reshape(-1).astype(jnp.int32)
    out = _sc_scatter_rows(idx, norm_tab)
    return out.reshape(seq.shape[0], seq.shape[1], _HIDDEN)
